# baseline (speedup 1.0000x reference)
import functools

import jax
import jax.numpy as jnp
from jax import lax
from jax.experimental import pallas as pl
from jax.experimental.pallas import tpu as pltpu
from jax.experimental.pallas import tpu_sc as plsc

B = 16384
CONT_DIM = 256
OUT_DIM = 256
HALF = 128
D1, D2, D3, D4 = 64, 32, 16, 32

NC, NS = 2, 16
NW = NC * NS
BPW = B // NW


def _sc_gather_small(idx4, E2, E4, E3):
    mesh = plsc.VectorSubcoreMesh(core_axis_name="c", subcore_axis_name="s")
    f32 = jnp.float32

    @functools.partial(
        pl.kernel,
        out_type=jax.ShapeDtypeStruct((B, 128), f32),
        mesh=mesh,
        compiler_params=pltpu.CompilerParams(use_tc_tiling_on_sc=False),
        scratch_types=[
            pltpu.VMEM((4, BPW), jnp.int32),
            pltpu.VMEM((BPW, D2), f32),
            pltpu.VMEM((BPW, D4), f32),
            pltpu.VMEM((BPW, D3), f32),
            pltpu.SemaphoreType.DMA,
            pltpu.SemaphoreType.DMA,
            pltpu.SemaphoreType.DMA,
            pltpu.SemaphoreType.DMA,
            pltpu.SemaphoreType.DMA,
        ],
    )
    def k(idx_h, e2_h, e4_h, e3_h, oc_h, idxv, r2, r4, r3,
          si, s2, s4, s3, sw):
        wid = lax.axis_index("s") * NC + lax.axis_index("c")
        base = wid * BPW
        sl = pl.ds(base, BPW)
        pltpu.async_copy(idx_h.at[:, sl], idxv, si).wait()
        g2 = pltpu.async_copy(e2_h.at[idxv.at[1]], r2, s2)
        g4 = pltpu.async_copy(e4_h.at[idxv.at[2]], r4, s4)
        g3 = pltpu.async_copy(e3_h.at[idxv.at[3]], r3, s3)
        g2.wait()
        w2 = pltpu.async_copy(r2, oc_h.at[sl, pl.ds(0, D2)], sw)
        g4.wait()
        w4 = pltpu.async_copy(r4, oc_h.at[sl, pl.ds(D2, D4)], sw)
        g3.wait()
        w3c = pltpu.async_copy(r3, oc_h.at[sl, pl.ds(D2 + D4, D3)], sw)
        w2.wait()
        w4.wait()
        w3c.wait()

    return k(idx4, E2, E4, E3)


def _sc_gather_big(idx3, E1p):
    mesh = plsc.VectorSubcoreMesh(core_axis_name="c", subcore_axis_name="s")
    f32 = jnp.float32

    @functools.partial(
        pl.kernel,
        out_type=jax.ShapeDtypeStruct((B, 128), f32),
        mesh=mesh,
        compiler_params=pltpu.CompilerParams(use_tc_tiling_on_sc=False),
        scratch_types=[
            pltpu.VMEM((4, BPW), jnp.int32),
            pltpu.VMEM((BPW, 128), f32),
            pltpu.SemaphoreType.DMA,
            pltpu.SemaphoreType.DMA,
            pltpu.SemaphoreType.DMA,
        ],
    )
    def k(idx_h, e1_h, o1_h, idxv, r1, si, s1, sw):
        wid = lax.axis_index("s") * NC + lax.axis_index("c")
        base = wid * BPW
        sl = pl.ds(base, BPW)
        pltpu.async_copy(idx_h.at[:, sl], idxv, si).wait()
        pltpu.async_copy(e1_h.at[idxv.at[0]], r1, s1).wait()
        pltpu.async_copy(r1, o1_h.at[sl], sw).wait()

    return k(idx3, E1p)


def _tc_body(x_ref, o1_ref, comb_ref, wc_ref, bc_ref,
             w0_ref, w1s_ref, wcat_ref, bf_ref, out_ref):
    f32 = jnp.float32
    i32 = jnp.int32
    bf16 = jnp.bfloat16
    bm = x_ref.shape[0]

    cont = jnp.dot(x_ref[...].astype(bf16), wc_ref[...],
                   preferred_element_type=f32)
    cont = cont + bc_ref[...]
    acc = jnp.dot(cont.astype(bf16), w0_ref[...], preferred_element_type=f32)

    acc = acc + jnp.dot(o1_ref[...].astype(bf16), w1s_ref[...],
                        preferred_element_type=f32)
    lt80 = lax.broadcasted_iota(i32, (bm, 128), 1) < (D2 + D4 + D3)
    combz = jnp.where(lt80, comb_ref[...], 0.0).astype(bf16)
    acc = acc + jnp.dot(combz, wcat_ref[...], preferred_element_type=f32)
    out_ref[...] = acc + bf_ref[...]


def kernel(continuous_inputs, pitcher_id, pitch_type, batter_side,
           game_situation, W_cont, b_cont, E_pitcher_id, E_pitch_type,
           E_batter_side, E_game_situation, W_final, b_final):
    i32 = jnp.int32
    pid = pitcher_id.astype(i32)
    idx4 = jnp.stack([pid, pitch_type.astype(i32),
                      game_situation.astype(i32), batter_side.astype(i32)])
    comb = _sc_gather_small(idx4, E_pitch_type, E_game_situation,
                            E_batter_side)
    eyepad = jnp.eye(D1, 128, dtype=jnp.float32)
    E1p = jnp.dot(E_pitcher_id, eyepad, precision=jax.lax.Precision.HIGHEST)
    o1p = _sc_gather_big(idx4, E1p)

    bf16 = jnp.bfloat16
    w0 = W_final[:HALF].astype(bf16)
    w1 = W_final[HALF:HALF + D1]
    w1s = jnp.concatenate(
        [w1, jnp.zeros((128 - D1, OUT_DIM), jnp.float32)], axis=0).astype(bf16)
    wcat = jnp.concatenate(
        [W_final[HALF + D1:HALF + D1 + D2], W_final[HALF + D1 + D2 + D3:],
         W_final[HALF + D1 + D2:HALF + D1 + D2 + D3],
         jnp.zeros((128 - D2 - D4 - D3, OUT_DIM), jnp.float32)],
        axis=0).astype(bf16)
    bc = b_cont.reshape(1, HALF)
    bf = b_final.reshape(1, OUT_DIM)

    BM = 1024
    grid = (B // BM,)
    row = lambda i: (i, 0)
    full = lambda i: (0, 0)
    out = pl.pallas_call(
        _tc_body,
        grid=grid,
        in_specs=[
            pl.BlockSpec((BM, CONT_DIM), row),
            pl.BlockSpec((BM, 128), row),
            pl.BlockSpec((BM, 128), row),
            pl.BlockSpec((CONT_DIM, HALF), full),
            pl.BlockSpec((1, HALF), full),
            pl.BlockSpec((HALF, OUT_DIM), full),
            pl.BlockSpec((128, OUT_DIM), full),
            pl.BlockSpec((128, OUT_DIM), full),
            pl.BlockSpec((1, OUT_DIM), full),
        ],
        out_specs=pl.BlockSpec((BM, OUT_DIM), row),
        out_shape=jax.ShapeDtypeStruct((B, OUT_DIM), jnp.float32),
        compiler_params=pltpu.CompilerParams(
            dimension_semantics=("arbitrary",),
        ),
    )(continuous_inputs, o1p, comb, W_cont.astype(bf16), bc, w0,
      w1s, wcat, bf)
    return out


# speedup vs baseline: 1.3692x; 1.3692x over previous
import functools

import jax
import jax.numpy as jnp
from jax import lax
from jax.experimental import pallas as pl
from jax.experimental.pallas import tpu as pltpu
from jax.experimental.pallas import tpu_sc as plsc

B = 16384
CONT_DIM = 256
OUT_DIM = 256
HALF = 128
D1, D2, D3, D4 = 64, 32, 16, 32

NC, NS = 2, 16
NW = NC * NS
BPW = B // NW


def _sc_gather_small(idx4, E2, E4, E3):
    mesh = plsc.VectorSubcoreMesh(core_axis_name="c", subcore_axis_name="s")
    f32 = jnp.float32

    @functools.partial(
        pl.kernel,
        out_type=jax.ShapeDtypeStruct((B, 128), f32),
        mesh=mesh,
        compiler_params=pltpu.CompilerParams(use_tc_tiling_on_sc=False),
        scratch_types=[
            pltpu.VMEM((4, BPW), jnp.int32),
            pltpu.VMEM((BPW, D2), f32),
            pltpu.VMEM((BPW, D4), f32),
            pltpu.VMEM((BPW, D3), f32),
            pltpu.SemaphoreType.DMA,
            pltpu.SemaphoreType.DMA,
            pltpu.SemaphoreType.DMA,
            pltpu.SemaphoreType.DMA,
            pltpu.SemaphoreType.DMA,
        ],
    )
    def k(idx_h, e2_h, e4_h, e3_h, oc_h, idxv, r2, r4, r3,
          si, s2, s4, s3, sw):
        wid = lax.axis_index("s") * NC + lax.axis_index("c")
        base = wid * BPW
        sl = pl.ds(base, BPW)
        pltpu.async_copy(idx_h.at[:, sl], idxv, si).wait()
        g2 = pltpu.async_copy(e2_h.at[idxv.at[1]], r2, s2)
        g4 = pltpu.async_copy(e4_h.at[idxv.at[2]], r4, s4)
        g3 = pltpu.async_copy(e3_h.at[idxv.at[3]], r3, s3)
        g2.wait()
        w2 = pltpu.async_copy(r2, oc_h.at[sl, pl.ds(0, D2)], sw)
        g4.wait()
        w4 = pltpu.async_copy(r4, oc_h.at[sl, pl.ds(D2, D4)], sw)
        g3.wait()
        w3c = pltpu.async_copy(r3, oc_h.at[sl, pl.ds(D2 + D4, D3)], sw)
        w2.wait()
        w4.wait()
        w3c.wait()

    return k(idx4, E2, E4, E3)


def _sc_gather_big(idx3, E1p):
    mesh = plsc.VectorSubcoreMesh(core_axis_name="c", subcore_axis_name="s")
    f32 = jnp.float32

    @functools.partial(
        pl.kernel,
        out_type=jax.ShapeDtypeStruct((B, 128), f32),
        mesh=mesh,
        compiler_params=pltpu.CompilerParams(use_tc_tiling_on_sc=False),
        scratch_types=[
            pltpu.VMEM((4, BPW), jnp.int32),
            pltpu.VMEM((BPW, 128), f32),
            pltpu.SemaphoreType.DMA,
            pltpu.SemaphoreType.DMA,
            pltpu.SemaphoreType.DMA,
        ],
    )
    def k(idx_h, e1_h, o1_h, idxv, r1, si, s1, sw):
        wid = lax.axis_index("s") * NC + lax.axis_index("c")
        base = wid * BPW
        sl = pl.ds(base, BPW)
        pltpu.async_copy(idx_h.at[:, sl], idxv, si).wait()
        pltpu.async_copy(e1_h.at[idxv.at[0]], r1, s1).wait()
        pltpu.async_copy(r1, o1_h.at[sl], sw).wait()

    return k(idx3, E1p)


def _tc_body(x_ref, o1_ref, comb_ref, wc_ref, bc_ref,
             w0_ref, w1s_ref, wcat_ref, bf_ref, out_ref):
    f32 = jnp.float32
    i32 = jnp.int32
    bf16 = jnp.bfloat16
    bm = x_ref.shape[0]

    cont = jnp.dot(x_ref[...].astype(bf16), wc_ref[...],
                   preferred_element_type=f32)
    cont = cont + bc_ref[...]
    acc = jnp.dot(cont.astype(bf16), w0_ref[...], preferred_element_type=f32)

    acc = acc + jnp.dot(o1_ref[...].astype(bf16), w1s_ref[...],
                        preferred_element_type=f32)
    lt80 = lax.broadcasted_iota(i32, (bm, 128), 1) < (D2 + D4 + D3)
    combz = jnp.where(lt80, comb_ref[...], 0.0).astype(bf16)
    acc = acc + jnp.dot(combz, wcat_ref[...], preferred_element_type=f32)
    out_ref[...] = acc + bf_ref[...]


def kernel(continuous_inputs, pitcher_id, pitch_type, batter_side,
           game_situation, W_cont, b_cont, E_pitcher_id, E_pitch_type,
           E_batter_side, E_game_situation, W_final, b_final):
    i32 = jnp.int32
    pid = pitcher_id.astype(i32)
    bidx = (batter_side.astype(i32)
            + D3 * (jnp.arange(B, dtype=i32) & 511))
    idx4 = jnp.stack([pid, pitch_type.astype(i32),
                      game_situation.astype(i32), bidx])
    E3r = jnp.tile(E_batter_side, (512, 1))
    comb = _sc_gather_small(idx4, E_pitch_type, E_game_situation, E3r)
    eyepad = jnp.eye(D1, 128, dtype=jnp.float32)
    E1p = jnp.dot(E_pitcher_id, eyepad, precision=jax.lax.Precision.HIGHEST)
    o1p = _sc_gather_big(idx4, E1p)

    bf16 = jnp.bfloat16
    w0 = W_final[:HALF].astype(bf16)
    w1 = W_final[HALF:HALF + D1]
    w1s = jnp.concatenate(
        [w1, jnp.zeros((128 - D1, OUT_DIM), jnp.float32)], axis=0).astype(bf16)
    wcat = jnp.concatenate(
        [W_final[HALF + D1:HALF + D1 + D2], W_final[HALF + D1 + D2 + D3:],
         W_final[HALF + D1 + D2:HALF + D1 + D2 + D3],
         jnp.zeros((128 - D2 - D4 - D3, OUT_DIM), jnp.float32)],
        axis=0).astype(bf16)
    bc = b_cont.reshape(1, HALF)
    bf = b_final.reshape(1, OUT_DIM)

    BM = 1024
    grid = (B // BM,)
    row = lambda i: (i, 0)
    full = lambda i: (0, 0)
    out = pl.pallas_call(
        _tc_body,
        grid=grid,
        in_specs=[
            pl.BlockSpec((BM, CONT_DIM), row),
            pl.BlockSpec((BM, 128), row),
            pl.BlockSpec((BM, 128), row),
            pl.BlockSpec((CONT_DIM, HALF), full),
            pl.BlockSpec((1, HALF), full),
            pl.BlockSpec((HALF, OUT_DIM), full),
            pl.BlockSpec((128, OUT_DIM), full),
            pl.BlockSpec((128, OUT_DIM), full),
            pl.BlockSpec((1, OUT_DIM), full),
        ],
        out_specs=pl.BlockSpec((BM, OUT_DIM), row),
        out_shape=jax.ShapeDtypeStruct((B, OUT_DIM), jnp.float32),
        compiler_params=pltpu.CompilerParams(
            dimension_semantics=("arbitrary",),
        ),
    )(continuous_inputs, o1p, comb, W_cont.astype(bf16), bc, w0,
      w1s, wcat, bf)
    return out


# speedup vs baseline: 1.3907x; 1.0157x over previous
import functools

import jax
import jax.numpy as jnp
from jax import lax
from jax.experimental import pallas as pl
from jax.experimental.pallas import tpu as pltpu
from jax.experimental.pallas import tpu_sc as plsc

B = 16384
CONT_DIM = 256
OUT_DIM = 256
HALF = 128
D1, D2, D3, D4 = 64, 32, 16, 32

NC, NS = 2, 16
NW = NC * NS
BPW = B // NW


def _sc_gather(idx4, E1p, E2, E4, E3):
    mesh = plsc.VectorSubcoreMesh(core_axis_name="c", subcore_axis_name="s")
    f32 = jnp.float32

    @functools.partial(
        pl.kernel,
        out_type=(
            jax.ShapeDtypeStruct((B, 128), f32),
            jax.ShapeDtypeStruct((B, 128), f32),
        ),
        mesh=mesh,
        compiler_params=pltpu.CompilerParams(use_tc_tiling_on_sc=False),
        scratch_types=[
            pltpu.VMEM((4, BPW), jnp.int32),
            pltpu.VMEM((BPW, 128), f32),
            pltpu.VMEM((BPW, D2), f32),
            pltpu.VMEM((BPW, D4), f32),
            pltpu.VMEM((BPW, D3), f32),
            pltpu.SemaphoreType.DMA,
            pltpu.SemaphoreType.DMA,
            pltpu.SemaphoreType.DMA,
            pltpu.SemaphoreType.DMA,
            pltpu.SemaphoreType.DMA,
            pltpu.SemaphoreType.DMA,
        ],
    )
    def k(idx_h, e1_h, e2_h, e4_h, e3_h, o1_h, oc_h,
          idxv, r1, r2, r4, r3, si, s1, s2, s4, s3, sw):
        wid = lax.axis_index("s") * NC + lax.axis_index("c")
        base = wid * BPW
        sl = pl.ds(base, BPW)
        pltpu.async_copy(idx_h.at[:, sl], idxv, si).wait()
        g1 = pltpu.async_copy(e1_h.at[idxv.at[0]], r1, s1)
        g2 = pltpu.async_copy(e2_h.at[idxv.at[1]], r2, s2)
        g4 = pltpu.async_copy(e4_h.at[idxv.at[2]], r4, s4)
        g3 = pltpu.async_copy(e3_h.at[idxv.at[3]], r3, s3)
        g2.wait()
        w2 = pltpu.async_copy(r2, oc_h.at[sl, pl.ds(0, D2)], sw)
        g4.wait()
        w4 = pltpu.async_copy(r4, oc_h.at[sl, pl.ds(D2, D4)], sw)
        g3.wait()
        w3c = pltpu.async_copy(r3, oc_h.at[sl, pl.ds(D2 + D4, D3)], sw)
        g1.wait()
        w1 = pltpu.async_copy(r1, o1_h.at[sl], sw)
        w2.wait()
        w4.wait()
        w3c.wait()
        w1.wait()

    return k(idx4, E1p, E2, E4, E3)


def _tc_body(x_ref, o1_ref, comb_ref, wc_ref, bc_ref,
             w0_ref, w1s_ref, wcat_ref, bf_ref, out_ref):
    f32 = jnp.float32
    i32 = jnp.int32
    bf16 = jnp.bfloat16
    bm = x_ref.shape[0]

    cont = jnp.dot(x_ref[...].astype(bf16), wc_ref[...],
                   preferred_element_type=f32)
    cont = cont + bc_ref[...]
    acc = jnp.dot(cont.astype(bf16), w0_ref[...], preferred_element_type=f32)

    acc = acc + jnp.dot(o1_ref[...].astype(bf16), w1s_ref[...],
                        preferred_element_type=f32)
    lt80 = lax.broadcasted_iota(i32, (bm, 128), 1) < (D2 + D4 + D3)
    combz = jnp.where(lt80, comb_ref[...], 0.0).astype(bf16)
    acc = acc + jnp.dot(combz, wcat_ref[...], preferred_element_type=f32)
    out_ref[...] = acc + bf_ref[...]


def kernel(continuous_inputs, pitcher_id, pitch_type, batter_side,
           game_situation, W_cont, b_cont, E_pitcher_id, E_pitch_type,
           E_batter_side, E_game_situation, W_final, b_final):
    i32 = jnp.int32
    pid = pitcher_id.astype(i32)
    bidx = (batter_side.astype(i32)
            + D3 * (jnp.arange(B, dtype=i32) & 511))
    idx4 = jnp.stack([pid, pitch_type.astype(i32),
                      game_situation.astype(i32), bidx])
    E3r = jnp.tile(E_batter_side, (512, 1))
    eyepad = jnp.eye(D1, 128, dtype=jnp.float32)
    E1p = jnp.dot(E_pitcher_id, eyepad, precision=jax.lax.Precision.HIGHEST)
    o1p, comb = _sc_gather(idx4, E1p, E_pitch_type, E_game_situation, E3r)

    bf16 = jnp.bfloat16
    w0 = W_final[:HALF].astype(bf16)
    w1 = W_final[HALF:HALF + D1]
    w1s = jnp.concatenate(
        [w1, jnp.zeros((128 - D1, OUT_DIM), jnp.float32)], axis=0).astype(bf16)
    wcat = jnp.concatenate(
        [W_final[HALF + D1:HALF + D1 + D2], W_final[HALF + D1 + D2 + D3:],
         W_final[HALF + D1 + D2:HALF + D1 + D2 + D3],
         jnp.zeros((128 - D2 - D4 - D3, OUT_DIM), jnp.float32)],
        axis=0).astype(bf16)
    bc = b_cont.reshape(1, HALF)
    bf = b_final.reshape(1, OUT_DIM)

    BM = 1024
    grid = (B // BM,)
    row = lambda i: (i, 0)
    full = lambda i: (0, 0)
    out = pl.pallas_call(
        _tc_body,
        grid=grid,
        in_specs=[
            pl.BlockSpec((BM, CONT_DIM), row),
            pl.BlockSpec((BM, 128), row),
            pl.BlockSpec((BM, 128), row),
            pl.BlockSpec((CONT_DIM, HALF), full),
            pl.BlockSpec((1, HALF), full),
            pl.BlockSpec((HALF, OUT_DIM), full),
            pl.BlockSpec((128, OUT_DIM), full),
            pl.BlockSpec((128, OUT_DIM), full),
            pl.BlockSpec((1, OUT_DIM), full),
        ],
        out_specs=pl.BlockSpec((BM, OUT_DIM), row),
        out_shape=jax.ShapeDtypeStruct((B, OUT_DIM), jnp.float32),
        compiler_params=pltpu.CompilerParams(
            dimension_semantics=("arbitrary",),
        ),
    )(continuous_inputs, o1p, comb, W_cont.astype(bf16), bc, w0,
      w1s, wcat, bf)
    return out


# speedup vs baseline: 1.4432x; 1.0377x over previous
import functools

import jax
import jax.numpy as jnp
from jax import lax
from jax.experimental import pallas as pl
from jax.experimental.pallas import tpu as pltpu
from jax.experimental.pallas import tpu_sc as plsc

B = 16384
CONT_DIM = 256
OUT_DIM = 256
HALF = 128
D1, D2, D3, D4 = 64, 32, 16, 32

NC, NS = 2, 16
NW = NC * NS
BPW = B // NW


def _sc_gather(idx4, E1p, E2, E4, E3):
    mesh = plsc.VectorSubcoreMesh(core_axis_name="c", subcore_axis_name="s")
    f32 = jnp.float32

    @functools.partial(
        pl.kernel,
        out_type=(
            jax.ShapeDtypeStruct((B, 128), f32),
            jax.ShapeDtypeStruct((B, 128), f32),
        ),
        mesh=mesh,
        compiler_params=pltpu.CompilerParams(use_tc_tiling_on_sc=False),
        scratch_types=[
            pltpu.VMEM((4, BPW), jnp.int32),
            pltpu.VMEM((BPW, 128), f32),
            pltpu.VMEM((BPW, D2), f32),
            pltpu.VMEM((BPW, D4), f32),
            pltpu.VMEM((BPW, D3), f32),
            pltpu.SemaphoreType.DMA,
            pltpu.SemaphoreType.DMA,
            pltpu.SemaphoreType.DMA,
            pltpu.SemaphoreType.DMA,
            pltpu.SemaphoreType.DMA,
            pltpu.SemaphoreType.DMA,
        ],
    )
    def k(idx_h, e1_h, e2_h, e4_h, e3_h, o1_h, oc_h,
          idxv, r1, r2, r4, r3, si, s1, s2, s4, s3, sw):
        wid = lax.axis_index("s") * NC + lax.axis_index("c")
        base = wid * BPW
        sl = pl.ds(base, BPW)
        pltpu.async_copy(idx_h.at[:, sl], idxv, si).wait()
        g1 = pltpu.async_copy(e1_h.at[idxv.at[0]], r1, s1)
        g2 = pltpu.async_copy(e2_h.at[idxv.at[1]], r2, s2)
        g4 = pltpu.async_copy(e4_h.at[idxv.at[2]], r4, s4)
        g3 = pltpu.async_copy(e3_h.at[idxv.at[3]], r3, s3)
        g2.wait()
        w2 = pltpu.async_copy(r2, oc_h.at[sl, pl.ds(0, D2)], sw)
        g4.wait()
        w4 = pltpu.async_copy(r4, oc_h.at[sl, pl.ds(D2, D4)], sw)
        g3.wait()
        w3c = pltpu.async_copy(r3, oc_h.at[sl, pl.ds(D2 + D4, D3)], sw)
        g1.wait()
        w1 = pltpu.async_copy(r1, o1_h.at[sl], sw)
        w2.wait()
        w4.wait()
        w3c.wait()
        w1.wait()

    return k(idx4, E1p, E2, E4, E3)


def _tc_body(x_ref, o1_ref, comb_ref, wc_ref, bc_ref,
             w0_ref, w1s_ref, wcat_ref, bf_ref, out_ref):
    f32 = jnp.float32
    i32 = jnp.int32
    bf16 = jnp.bfloat16
    bm = x_ref.shape[0]

    cont = jnp.dot(x_ref[...].astype(bf16), wc_ref[...],
                   preferred_element_type=f32)
    cont = cont + bc_ref[...]
    acc = jnp.dot(cont.astype(bf16), w0_ref[...], preferred_element_type=f32)

    acc = acc + jnp.dot(o1_ref[...].astype(bf16), w1s_ref[...],
                        preferred_element_type=f32)
    lt80 = lax.broadcasted_iota(i32, (bm, 128), 1) < (D2 + D4 + D3)
    combz = jnp.where(lt80, comb_ref[...], 0.0).astype(bf16)
    acc = acc + jnp.dot(combz, wcat_ref[...], preferred_element_type=f32)
    out_ref[...] = acc + bf_ref[...]


def kernel(continuous_inputs, pitcher_id, pitch_type, batter_side,
           game_situation, W_cont, b_cont, E_pitcher_id, E_pitch_type,
           E_batter_side, E_game_situation, W_final, b_final):
    i32 = jnp.int32
    pid = pitcher_id.astype(i32)
    bidx = (batter_side.astype(i32)
            + D3 * (jnp.arange(B, dtype=i32) & 511))
    idx4 = jnp.stack([pid, pitch_type.astype(i32),
                      game_situation.astype(i32), bidx])
    E3r = jnp.tile(E_batter_side, (512, 1))
    eyepad = jnp.eye(D1, 128, dtype=jnp.float32)
    E1p = jnp.dot(E_pitcher_id, eyepad, precision=jax.lax.Precision.HIGHEST)
    o1p, comb = _sc_gather(idx4, E1p, E_pitch_type, E_game_situation, E3r)

    bf16 = jnp.bfloat16
    w0 = W_final[:HALF].astype(bf16)
    w1 = W_final[HALF:HALF + D1]
    w1s = jnp.concatenate(
        [w1, jnp.zeros((128 - D1, OUT_DIM), jnp.float32)], axis=0).astype(bf16)
    wcat = jnp.concatenate(
        [W_final[HALF + D1:HALF + D1 + D2], W_final[HALF + D1 + D2 + D3:],
         W_final[HALF + D1 + D2:HALF + D1 + D2 + D3],
         jnp.zeros((128 - D2 - D4 - D3, OUT_DIM), jnp.float32)],
        axis=0).astype(bf16)
    bc = b_cont.reshape(1, HALF)
    bf = b_final.reshape(1, OUT_DIM)

    BM = 2048
    grid = (B // BM,)
    row = lambda i: (i, 0)
    full = lambda i: (0, 0)
    out = pl.pallas_call(
        _tc_body,
        grid=grid,
        in_specs=[
            pl.BlockSpec((BM, CONT_DIM), row),
            pl.BlockSpec((BM, 128), row),
            pl.BlockSpec((BM, 128), row),
            pl.BlockSpec((CONT_DIM, HALF), full),
            pl.BlockSpec((1, HALF), full),
            pl.BlockSpec((HALF, OUT_DIM), full),
            pl.BlockSpec((128, OUT_DIM), full),
            pl.BlockSpec((128, OUT_DIM), full),
            pl.BlockSpec((1, OUT_DIM), full),
        ],
        out_specs=pl.BlockSpec((BM, OUT_DIM), row),
        out_shape=jax.ShapeDtypeStruct((B, OUT_DIM), jnp.float32),
        compiler_params=pltpu.CompilerParams(
            dimension_semantics=("arbitrary",),
        ),
    )(continuous_inputs, o1p, comb, W_cont.astype(bf16), bc, w0,
      w1s, wcat, bf)
    return out


# speedup vs baseline: 1.4572x; 1.0097x over previous
import functools

import jax
import jax.numpy as jnp
from jax import lax
from jax.experimental import pallas as pl
from jax.experimental.pallas import tpu as pltpu
from jax.experimental.pallas import tpu_sc as plsc

B = 16384
CONT_DIM = 256
OUT_DIM = 256
HALF = 128
D1, D2, D3, D4 = 64, 32, 16, 32

NC, NS = 2, 16
NW = NC * NS
BPW = B // NW


def _sc_gather(idx4, E1p, E2, E4, E3):
    mesh = plsc.VectorSubcoreMesh(core_axis_name="c", subcore_axis_name="s")
    f32 = jnp.float32

    @functools.partial(
        pl.kernel,
        out_type=(
            jax.ShapeDtypeStruct((B, 128), f32),
            jax.ShapeDtypeStruct((B, 128), f32),
        ),
        mesh=mesh,
        compiler_params=pltpu.CompilerParams(use_tc_tiling_on_sc=False),
        scratch_types=[
            pltpu.VMEM((4, BPW), jnp.int32),
            pltpu.VMEM((BPW, 128), f32),
            pltpu.VMEM((BPW, D2), f32),
            pltpu.VMEM((BPW, D4), f32),
            pltpu.VMEM((BPW, D3), f32),
            pltpu.SemaphoreType.DMA,
            pltpu.SemaphoreType.DMA,
            pltpu.SemaphoreType.DMA,
            pltpu.SemaphoreType.DMA,
            pltpu.SemaphoreType.DMA,
            pltpu.SemaphoreType.DMA,
        ],
    )
    def k(idx_h, e1_h, e2_h, e4_h, e3_h, o1_h, oc_h,
          idxv, r1, r2, r4, r3, si, s1, s2, s4, s3, sw):
        wid = lax.axis_index("s") * NC + lax.axis_index("c")
        base = wid * BPW
        sl = pl.ds(base, BPW)
        pltpu.async_copy(idx_h.at[:, sl], idxv, si).wait()
        g1 = pltpu.async_copy(e1_h.at[idxv.at[0]], r1, s1)
        g2 = pltpu.async_copy(e2_h.at[idxv.at[1]], r2, s2)
        g4 = pltpu.async_copy(e4_h.at[idxv.at[2]], r4, s4)
        g3 = pltpu.async_copy(e3_h.at[idxv.at[3]], r3, s3)
        g2.wait()
        w2 = pltpu.async_copy(r2, oc_h.at[sl, pl.ds(0, D2)], sw)
        g4.wait()
        w4 = pltpu.async_copy(r4, oc_h.at[sl, pl.ds(D2, D4)], sw)
        g3.wait()
        w3c = pltpu.async_copy(r3, oc_h.at[sl, pl.ds(D2 + D4, D3)], sw)
        g1.wait()
        w1 = pltpu.async_copy(r1, o1_h.at[sl], sw)
        w2.wait()
        w4.wait()
        w3c.wait()
        w1.wait()

    return k(idx4, E1p, E2, E4, E3)


def _tc_body(x_ref, o1_ref, comb_ref, wc_ref, bc_ref,
             w0_ref, w1s_ref, wcat_ref, bf_ref, out_ref):
    f32 = jnp.float32
    i32 = jnp.int32
    bf16 = jnp.bfloat16
    bm = x_ref.shape[0]

    cont = jnp.dot(x_ref[...].astype(bf16), wc_ref[...],
                   preferred_element_type=f32)
    cont = cont + bc_ref[...]
    acc = jnp.dot(cont.astype(bf16), w0_ref[...], preferred_element_type=f32)

    acc = acc + jnp.dot(o1_ref[...].astype(bf16), w1s_ref[...],
                        preferred_element_type=f32)
    lt80 = lax.broadcasted_iota(i32, (bm, 128), 1) < (D2 + D4 + D3)
    combz = jnp.where(lt80, comb_ref[...], 0.0).astype(bf16)
    acc = acc + jnp.dot(combz, wcat_ref[...], preferred_element_type=f32)
    out_ref[...] = acc + bf_ref[...]


def kernel(continuous_inputs, pitcher_id, pitch_type, batter_side,
           game_situation, W_cont, b_cont, E_pitcher_id, E_pitch_type,
           E_batter_side, E_game_situation, W_final, b_final):
    i32 = jnp.int32
    pid = pitcher_id.astype(i32)
    bidx = (batter_side.astype(i32)
            + D3 * (jnp.arange(B, dtype=i32) & 511))
    idx4 = jnp.stack([pid, pitch_type.astype(i32),
                      game_situation.astype(i32), bidx])
    E3r = jnp.tile(E_batter_side, (512, 1))
    eyepad = jnp.eye(D1, 128, dtype=jnp.float32)
    E1p = jnp.dot(E_pitcher_id, eyepad, precision=jax.lax.Precision.HIGHEST)
    o1p, comb = _sc_gather(idx4, E1p, E_pitch_type, E_game_situation, E3r)

    bf16 = jnp.bfloat16
    w0 = W_final[:HALF].astype(bf16)
    w1 = W_final[HALF:HALF + D1]
    w1s = jnp.concatenate(
        [w1, jnp.zeros((128 - D1, OUT_DIM), jnp.float32)], axis=0).astype(bf16)
    wcat = jnp.concatenate(
        [W_final[HALF + D1:HALF + D1 + D2], W_final[HALF + D1 + D2 + D3:],
         W_final[HALF + D1 + D2:HALF + D1 + D2 + D3],
         jnp.zeros((128 - D2 - D4 - D3, OUT_DIM), jnp.float32)],
        axis=0).astype(bf16)
    bc = b_cont.reshape(1, HALF)
    bf = b_final.reshape(1, OUT_DIM)

    BM = 4096
    grid = (B // BM,)
    row = lambda i: (i, 0)
    full = lambda i: (0, 0)
    out = pl.pallas_call(
        _tc_body,
        grid=grid,
        in_specs=[
            pl.BlockSpec((BM, CONT_DIM), row),
            pl.BlockSpec((BM, 128), row),
            pl.BlockSpec((BM, 128), row),
            pl.BlockSpec((CONT_DIM, HALF), full),
            pl.BlockSpec((1, HALF), full),
            pl.BlockSpec((HALF, OUT_DIM), full),
            pl.BlockSpec((128, OUT_DIM), full),
            pl.BlockSpec((128, OUT_DIM), full),
            pl.BlockSpec((1, OUT_DIM), full),
        ],
        out_specs=pl.BlockSpec((BM, OUT_DIM), row),
        out_shape=jax.ShapeDtypeStruct((B, OUT_DIM), jnp.float32),
        compiler_params=pltpu.CompilerParams(
            dimension_semantics=("arbitrary",),
        ),
    )(continuous_inputs, o1p, comb, W_cont.astype(bf16), bc, w0,
      w1s, wcat, bf)
    return out
